# baseline (speedup 1.0000x reference)
import jax
import jax.numpy as jnp
from jax import lax
from jax.experimental import pallas as pl
from jax.experimental.pallas import tpu as pltpu
from jax.experimental.pallas import tpu_sc as plsc

N_FEATURES = 64
BATCH = 4096
SEQ_LEN = 200
VOCAB_ROWS = 100000
VPAD = 100096
ROW_W = 128

NC = 2
NS = 16
NW = NC * NS

C = 128
NTB = BATCH // C
G = SEQ_LEN * NTB // NW

VBLKS = VPAD // ROW_W
VBLK_FULL = VOCAB_ROWS // ROW_W
VTAIL = VOCAB_ROWS - VBLK_FULL * ROW_W
BLK_PER_SUB = (VBLKS + NS - 1) // NS


def _body(xt_ref, tblt_ref, post_ref, tail_ref, out_ref, tbl_rm, t_in, t_out, pos_v,
          idxb, gbuf, obuf, sem_p1, sem_idx, sem_gat, sem_o0, sem_o1):
    cid = lax.axis_index("c")
    sid = lax.axis_index("s")
    iota16 = lax.iota(jnp.int32, 16)
    my_rm = tbl_rm.at[cid]

    pltpu.sync_copy(post_ref, pos_v)

    def transpose_block(v0, width):
        @pl.loop(0, width)
        def _(v):
            vsplat = jnp.full((16,), 0, jnp.int32) + v
            for k in range(4):
                vec = plsc.load_gather(t_in, [iota16 + 16 * k, vsplat])
                t_out[v, pl.ds(16 * k, 16)] = vec
        pltpu.sync_copy(t_out.at[pl.ds(0, width)], my_rm.at[pl.ds(v0, width)])

    @pl.loop(0, BLK_PER_SUB)
    def _(g):
        blk = sid * BLK_PER_SUB + g

        @pl.when(blk < VBLK_FULL)
        def _():
            v0 = blk * ROW_W
            pltpu.sync_copy(tblt_ref.at[:, pl.ds(v0, ROW_W)], t_in)
            transpose_block(v0, ROW_W)

        @pl.when(blk == VBLK_FULL)
        def _():
            v0 = VBLK_FULL * ROW_W
            pltpu.sync_copy(tail_ref, t_out.at[pl.ds(0, VTAIL)])
            pltpu.sync_copy(t_out.at[pl.ds(0, VTAIL)], my_rm.at[pl.ds(v0, VTAIL)])

    plsc.subcore_barrier()

    wid = sid * NC + cid
    f0 = wid * G

    def chunk_sb(f):
        return f // NTB, lax.rem(f, NTB)

    def idx_copy(f, b):
        s, tb = chunk_sb(f)
        return pltpu.make_async_copy(xt_ref.at[s, pl.ds(tb * C, C)],
                                     idxb.at[b], sem_idx)

    def gat_copy(b):
        return pltpu.make_async_copy(my_rm.at[idxb.at[b]], gbuf.at[b], sem_gat)

    def out_copy(f, b):
        s, tb = chunk_sb(f)
        return pltpu.make_async_copy(
            obuf.at[b], out_ref.at[s, :, pl.ds(tb * C, C)],
            sem_o1 if b else sem_o0)

    idx_copy(f0, 0).start()
    idx_copy(f0, 0).wait()
    gat_copy(0).start()
    idx_copy(f0 + 1, 1).start()

    @pl.loop(0, G // 2)
    def _(go):
        for par in range(2):
            g = go * 2 + par
            f = f0 + g
            gat_copy(par).wait()

            @pl.when(g + 1 < G)
            def _():
                idx_copy(f + 1, 1 - par).wait()
                gat_copy(1 - par).start()

            @pl.when(g + 2 < G)
            def _():
                idx_copy(f + 2, par).start()

            @pl.when(g >= 2)
            def _():
                out_copy(f - 2, par).wait()

            s, _ = chunk_sb(f)
            gb = gbuf.at[par]
            ob = obuf.at[par]

            ssplat = jnp.full((16,), 0, jnp.int32) + s

            @pl.loop(0, N_FEATURES)
            def _(ff):
                fsplat = jnp.full((16,), 0, jnp.int32) + ff
                psplat = plsc.load_gather(pos_v, [fsplat, ssplat])
                for k in range(8):
                    vec = plsc.load_gather(gb, [iota16 + 16 * k, fsplat])
                    ob[ff, pl.ds(16 * k, 16)] = vec + psplat

            out_copy(f, par).start()

    out_copy(f0 + G - 2, 0).wait()
    out_copy(f0 + G - 1, 1).wait()


@jax.jit
def _embed(xt, tblt, post, tail):
    run = pl.kernel(
        _body,
        out_type=(
            jax.ShapeDtypeStruct((SEQ_LEN, N_FEATURES, BATCH), jnp.float32),
            jax.ShapeDtypeStruct((NC, VPAD, ROW_W), jnp.float32),
        ),
        mesh=plsc.VectorSubcoreMesh(core_axis_name="c", subcore_axis_name="s"),
        scratch_types=[
            pltpu.VMEM((N_FEATURES, ROW_W), jnp.float32),
            pltpu.VMEM((ROW_W, ROW_W), jnp.float32),
            pltpu.VMEM((N_FEATURES, SEQ_LEN), jnp.float32),
            pltpu.VMEM((2, C), jnp.int32),
            pltpu.VMEM((2, C, ROW_W), jnp.float32),
            pltpu.VMEM((2, N_FEATURES, C), jnp.float32),
            pltpu.SemaphoreType.DMA,
            pltpu.SemaphoreType.DMA,
            pltpu.SemaphoreType.DMA,
            pltpu.SemaphoreType.DMA,
            pltpu.SemaphoreType.DMA,
        ],
        compiler_params=pltpu.CompilerParams(
            use_tc_tiling_on_sc=True, needs_layout_passes=False),
    )
    out_t, _ = run(xt, tblt, post, tail)
    return out_t


def kernel(x, text_embedding_weight, pos_embedding):
    bs, seq_len = x.shape
    xt = x.T.astype(jnp.int32)
    tblt = text_embedding_weight.T
    post = pos_embedding.reshape(-1, pos_embedding.shape[-1])[:seq_len].T
    tail = jnp.pad(text_embedding_weight[VBLK_FULL * ROW_W:],
                   ((0, 0), (0, ROW_W - N_FEATURES)))
    out_t = _embed(xt, tblt, post, tail)
    return jnp.transpose(out_t, (2, 0, 1))


# speedup vs baseline: 1.7654x; 1.7654x over previous
import jax
import jax.numpy as jnp
from jax import lax
from jax.experimental import pallas as pl
from jax.experimental.pallas import tpu as pltpu
from jax.experimental.pallas import tpu_sc as plsc

N_FEATURES = 64
BATCH = 4096
SEQ_LEN = 200
VOCAB_ROWS = 100000
VPAD = 100096
ROW_W = 128

NC = 2
NS = 16
NW = NC * NS

C = 128
NTB = BATCH // C
G = SEQ_LEN * NTB // NW

VBLKS = VPAD // ROW_W
VBLK_FULL = VOCAB_ROWS // ROW_W
VTAIL = VOCAB_ROWS - VBLK_FULL * ROW_W
BLK_PER_SUB = (VBLKS + NS - 1) // NS


def _body(xt_ref, tblt_ref, post_ref, tail_ref, out_ref, tbl_rm, t_in, t_out, pos_v,
          idxb, gbuf, obuf, sem_p1, sem_idx, sem_gat, sem_o0, sem_o1):
    cid = lax.axis_index("c")
    sid = lax.axis_index("s")
    iota16 = lax.iota(jnp.int32, 16)
    my_rm = tbl_rm.at[cid]

    pltpu.sync_copy(post_ref, pos_v)

    def transpose_block(v0, width):
        @plsc.parallel_loop(0, N_FEATURES, unroll=4)
        def _(f):
            fsplat = jnp.full((16,), 0, jnp.int32) + f
            for k in range(8):
                vec = t_in[f, pl.ds(16 * k, 16)]
                plsc.store_scatter(t_out, [iota16 + 16 * k, fsplat], vec)
        pltpu.sync_copy(t_out.at[pl.ds(0, width), pl.ds(0, ROW_W)],
                        my_rm.at[pl.ds(v0, width)])

    @pl.loop(0, BLK_PER_SUB)
    def _(g):
        blk = sid * BLK_PER_SUB + g

        @pl.when(blk < VBLK_FULL)
        def _():
            v0 = blk * ROW_W
            pltpu.sync_copy(tblt_ref.at[:, pl.ds(v0, ROW_W)], t_in)
            transpose_block(v0, ROW_W)

        @pl.when(blk == VBLK_FULL)
        def _():
            v0 = VBLK_FULL * ROW_W
            pltpu.sync_copy(tail_ref, t_out.at[pl.ds(0, VTAIL), pl.ds(0, ROW_W)])
            pltpu.sync_copy(t_out.at[pl.ds(0, VTAIL), pl.ds(0, ROW_W)],
                            my_rm.at[pl.ds(v0, VTAIL)])

    plsc.subcore_barrier()

    wid = sid * NC + cid
    f0 = wid * G

    def chunk_sb(f):
        return f // NTB, lax.rem(f, NTB)

    def idx_copy(f, b):
        s, tb = chunk_sb(f)
        return pltpu.make_async_copy(xt_ref.at[s, pl.ds(tb * C, C)],
                                     idxb.at[b], sem_idx)

    def gat_copy(b):
        return pltpu.make_async_copy(my_rm.at[idxb.at[b]], gbuf.at[b], sem_gat)

    def out_copy(f, b):
        s, tb = chunk_sb(f)
        return pltpu.make_async_copy(
            obuf.at[b, :, pl.ds(0, C)], out_ref.at[s, :, pl.ds(tb * C, C)],
            sem_o1 if b else sem_o0)

    idx_copy(f0, 0).start()
    idx_copy(f0, 0).wait()
    gat_copy(0).start()
    idx_copy(f0 + 1, 1).start()

    @pl.loop(0, G // 2)
    def _(go):
        for par in range(2):
            g = go * 2 + par
            f = f0 + g
            gat_copy(par).wait()

            @pl.when(g + 1 < G)
            def _():
                idx_copy(f + 1, 1 - par).wait()
                gat_copy(1 - par).start()

            @pl.when(g + 2 < G)
            def _():
                idx_copy(f + 2, par).start()

            @pl.when(g >= 2)
            def _():
                out_copy(f - 2, par).wait()

            s, _ = chunk_sb(f)
            gb = gbuf.at[par]
            ob = obuf.at[par]

            ssplat = jnp.full((16,), 0, jnp.int32) + s
            pos_k = [plsc.load_gather(pos_v, [iota16 + 16 * k, ssplat])
                     for k in range(4)]

            @plsc.parallel_loop(0, C, unroll=4)
            def _(bb):
                bsplat = jnp.full((16,), 0, jnp.int32) + bb
                for k in range(4):
                    vec = gb[bb, pl.ds(16 * k, 16)]
                    plsc.store_scatter(ob, [iota16 + 16 * k, bsplat],
                                       vec + pos_k[k])

            out_copy(f, par).start()

    out_copy(f0 + G - 2, 0).wait()
    out_copy(f0 + G - 1, 1).wait()


@jax.jit
def _embed(xt, tblt, post, tail):
    run = pl.kernel(
        _body,
        out_type=(
            jax.ShapeDtypeStruct((SEQ_LEN, N_FEATURES, BATCH), jnp.float32),
            jax.ShapeDtypeStruct((NC, VPAD, ROW_W), jnp.float32),
        ),
        mesh=plsc.VectorSubcoreMesh(core_axis_name="c", subcore_axis_name="s"),
        scratch_types=[
            pltpu.VMEM((N_FEATURES, ROW_W), jnp.float32),
            pltpu.VMEM((ROW_W, ROW_W + 1), jnp.float32),
            pltpu.VMEM((N_FEATURES, SEQ_LEN), jnp.float32),
            pltpu.VMEM((2, C), jnp.int32),
            pltpu.VMEM((2, C, ROW_W), jnp.float32),
            pltpu.VMEM((2, N_FEATURES, C + 1), jnp.float32),
            pltpu.SemaphoreType.DMA,
            pltpu.SemaphoreType.DMA,
            pltpu.SemaphoreType.DMA,
            pltpu.SemaphoreType.DMA,
            pltpu.SemaphoreType.DMA,
        ],
        compiler_params=pltpu.CompilerParams(
            use_tc_tiling_on_sc=True, needs_layout_passes=False),
    )
    out_t, _ = run(xt, tblt, post, tail)
    return out_t


def kernel(x, text_embedding_weight, pos_embedding):
    bs, seq_len = x.shape
    xt = x.T.astype(jnp.int32)
    tblt = text_embedding_weight.T
    post = pos_embedding.reshape(-1, pos_embedding.shape[-1])[:seq_len].T
    tail = jnp.pad(text_embedding_weight[VBLK_FULL * ROW_W:],
                   ((0, 0), (0, ROW_W - N_FEATURES)))
    out_t = _embed(xt, tblt, post, tail)
    return jnp.transpose(out_t, (2, 0, 1))


# speedup vs baseline: 3.9219x; 2.2215x over previous
import jax
import jax.numpy as jnp
from jax import lax
from jax.experimental import pallas as pl
from jax.experimental.pallas import tpu as pltpu
from jax.experimental.pallas import tpu_sc as plsc

N_FEATURES = 64
BATCH = 4096
SEQ_LEN = 200
VOCAB_ROWS = 100000
VPAD = 100096
ROW_W = 128

NC = 2
NS = 16
NW = NC * NS

C = 128
NTB = BATCH // C
G = SEQ_LEN * NTB // NW

VBLKS = VPAD // ROW_W
VBLK_FULL = VOCAB_ROWS // ROW_W
VTAIL = VOCAB_ROWS - VBLK_FULL * ROW_W
BLK_PER_SUB = (VBLKS + NS - 1) // NS


def _body(xt_ref, tblt_ref, post_ref, tail_ref, out_ref, tbl_rm, t_in, t_out, pos_v,
          idxb, gbuf, obuf, sem_p1, sem_idx, sem_gat, sem_o0, sem_o1):
    cid = lax.axis_index("c")
    sid = lax.axis_index("s")
    iota16 = lax.iota(jnp.int32, 16)
    my_rm = tbl_rm.at[cid]

    pltpu.sync_copy(post_ref, pos_v)

    def transpose_block(v0):
        @plsc.parallel_loop(0, N_FEATURES, unroll=2)
        def _(i):
            d = lax.rem(i, 16)
            f0 = i - d
            frow = ((iota16 + d) & 15) + f0
            for w0 in range(0, ROW_W, 16):
                wcol = iota16 + w0
                vec = plsc.load_gather(t_in, [frow, wcol])
                plsc.store_scatter(t_out, [wcol, frow], vec)
        pltpu.sync_copy(t_out, my_rm.at[pl.ds(v0, ROW_W)])

    @pl.loop(0, BLK_PER_SUB)
    def _(g):
        blk = sid * BLK_PER_SUB + g

        @pl.when(blk < VBLK_FULL)
        def _():
            v0 = blk * ROW_W
            pltpu.sync_copy(tblt_ref.at[:, pl.ds(v0, ROW_W)], t_in)
            transpose_block(v0)

        @pl.when(blk == VBLK_FULL)
        def _():
            v0 = VBLK_FULL * ROW_W
            pltpu.sync_copy(tail_ref, t_out.at[pl.ds(0, VTAIL)])
            pltpu.sync_copy(t_out.at[pl.ds(0, VTAIL)],
                            my_rm.at[pl.ds(v0, VTAIL)])

    plsc.subcore_barrier()

    wid = sid * NC + cid
    f0 = wid * G

    def chunk_sb(f):
        return f // NTB, lax.rem(f, NTB)

    def idx_copy(f, b):
        s, tb = chunk_sb(f)
        return pltpu.make_async_copy(xt_ref.at[s, pl.ds(tb * C, C)],
                                     idxb.at[b], sem_idx)

    def gat_copy(b):
        return pltpu.make_async_copy(my_rm.at[idxb.at[b]], gbuf.at[b], sem_gat)

    def out_copy(f, b):
        s, tb = chunk_sb(f)
        return pltpu.make_async_copy(
            obuf.at[b], out_ref.at[s, :, pl.ds(tb * C, C)],
            sem_o1 if b else sem_o0)

    idx_copy(f0, 0).start()
    idx_copy(f0, 0).wait()
    gat_copy(0).start()
    idx_copy(f0 + 1, 1).start()

    @pl.loop(0, G // 2)
    def _(go):
        for par in range(2):
            g = go * 2 + par
            f = f0 + g
            gat_copy(par).wait()

            @pl.when(g + 1 < G)
            def _():
                idx_copy(f + 1, 1 - par).wait()
                gat_copy(1 - par).start()

            @pl.when(g + 2 < G)
            def _():
                idx_copy(f + 2, par).start()

            @pl.when(g >= 2)
            def _():
                out_copy(f - 2, par).wait()

            s, _ = chunk_sb(f)
            gb = gbuf.at[par]
            ob = obuf.at[par]
            ssplat = jnp.full((16,), 0, jnp.int32) + s

            @plsc.parallel_loop(0, N_FEATURES, unroll=2)
            def _(i):
                d = lax.rem(i, 16)
                f0 = i - d
                col = ((iota16 + d) & 15) + f0
                pvec = plsc.load_gather(pos_v, [ssplat, col])
                for b0 in range(0, C, 16):
                    row = iota16 + b0
                    vec = plsc.load_gather(gb, [row, col])
                    plsc.store_scatter(ob, [col, row], vec + pvec)

            out_copy(f, par).start()

    out_copy(f0 + G - 2, 0).wait()
    out_copy(f0 + G - 1, 1).wait()


@jax.jit
def _embed(xt, tblt, post, tail):
    run = pl.kernel(
        _body,
        out_type=(
            jax.ShapeDtypeStruct((SEQ_LEN, N_FEATURES, BATCH), jnp.float32),
            jax.ShapeDtypeStruct((NC, VPAD, ROW_W), jnp.float32),
        ),
        mesh=plsc.VectorSubcoreMesh(core_axis_name="c", subcore_axis_name="s"),
        scratch_types=[
            pltpu.VMEM((N_FEATURES, ROW_W), jnp.float32),
            pltpu.VMEM((ROW_W, ROW_W), jnp.float32),
            pltpu.VMEM((SEQ_LEN, N_FEATURES), jnp.float32),
            pltpu.VMEM((2, C), jnp.int32),
            pltpu.VMEM((2, C, ROW_W), jnp.float32),
            pltpu.VMEM((2, N_FEATURES, C), jnp.float32),
            pltpu.SemaphoreType.DMA,
            pltpu.SemaphoreType.DMA,
            pltpu.SemaphoreType.DMA,
            pltpu.SemaphoreType.DMA,
            pltpu.SemaphoreType.DMA,
        ],
        compiler_params=pltpu.CompilerParams(
            use_tc_tiling_on_sc=True, needs_layout_passes=False),
    )
    out_t, _ = run(xt, tblt, post, tail)
    return out_t


def kernel(x, text_embedding_weight, pos_embedding):
    bs, seq_len = x.shape
    xt = x.T.astype(jnp.int32)
    tblt = text_embedding_weight.T
    post = pos_embedding.reshape(-1, pos_embedding.shape[-1])[:seq_len]
    tail = jnp.pad(text_embedding_weight[VBLK_FULL * ROW_W:],
                   ((0, 0), (0, ROW_W - N_FEATURES)))
    out_t = _embed(xt, tblt, post, tail)
    return jnp.transpose(out_t, (2, 0, 1))


# speedup vs baseline: 4.5513x; 1.1605x over previous
import jax
import jax.numpy as jnp
from jax import lax
from jax.experimental import pallas as pl
from jax.experimental.pallas import tpu as pltpu
from jax.experimental.pallas import tpu_sc as plsc

N_FEATURES = 64
BATCH = 4096
SEQ_LEN = 200
VOCAB_ROWS = 100000
VPAD = 100096
ROW_W = 128

NC = 2
NS = 16
NW = NC * NS

C = 128
NTB = BATCH // C
G = SEQ_LEN * NTB // NW

VBLKS = VPAD // ROW_W
VBLK_FULL = VOCAB_ROWS // ROW_W
VTAIL = VOCAB_ROWS - VBLK_FULL * ROW_W
BLK_PER_SUB = (VBLKS + NS - 1) // NS


def _body(xt_ref, tblt_ref, post_ref, tail_ref, out_ref, tbl_rm, t_in, t_out, pos_v,
          idxb, gbuf, obuf, sem_g1, sem_idx, sem_gat, sem_o0, sem_o1):
    cid = lax.axis_index("c")
    sid = lax.axis_index("s")
    iota16 = lax.iota(jnp.int32, 16)
    my_rm = tbl_rm.at[cid]

    pltpu.sync_copy(post_ref, pos_v)

    def transpose_block(v0):
        @plsc.parallel_loop(0, N_FEATURES, unroll=2)
        def _(i):
            d = lax.rem(i, 16)
            f0 = i - d
            frow = ((iota16 + d) & 15) + f0
            for w0 in range(0, ROW_W, 16):
                wcol = iota16 + w0
                vec = plsc.load_gather(t_in, [frow, wcol])
                plsc.store_scatter(t_out, [wcol, frow], vec)
        pltpu.sync_copy(t_out, my_rm.at[pl.ds(v0, ROW_W)])

    @pl.loop(0, BLK_PER_SUB)
    def _(g):
        blk = sid * BLK_PER_SUB + g

        @pl.when(blk < VBLK_FULL)
        def _():
            v0 = blk * ROW_W
            pltpu.sync_copy(tblt_ref.at[:, pl.ds(v0, ROW_W)], t_in)
            transpose_block(v0)

        @pl.when(blk == VBLK_FULL)
        def _():
            v0 = VBLK_FULL * ROW_W
            pltpu.sync_copy(tail_ref, t_out.at[pl.ds(0, VTAIL)])
            pltpu.sync_copy(t_out.at[pl.ds(0, VTAIL)],
                            my_rm.at[pl.ds(v0, VTAIL)])

    plsc.subcore_barrier()

    wid = sid * NC + cid
    f0 = wid * G

    def chunk_sb(f):
        return f // NTB, lax.rem(f, NTB)

    def idx_copy(f, b):
        s, tb = chunk_sb(f)
        return pltpu.make_async_copy(xt_ref.at[s, pl.ds(tb * C, C)],
                                     idxb.at[b], sem_idx)

    def gat_copy(b):
        return pltpu.make_async_copy(my_rm.at[idxb.at[b]], gbuf.at[b],
                                     sem_g1 if b else sem_gat)

    def out_copy(f, b):
        s, tb = chunk_sb(f)
        return pltpu.make_async_copy(
            obuf.at[b], out_ref.at[s, :, pl.ds(tb * C, C)],
            sem_o1 if b else sem_o0)

    idx_copy(f0, 0).start()
    idx_copy(f0, 0).wait()
    gat_copy(0).start()
    idx_copy(f0 + 1, 1).start()

    @pl.loop(0, G // 2)
    def _(go):
        for par in range(2):
            g = go * 2 + par
            f = f0 + g

            @pl.when(g + 1 < G)
            def _():
                idx_copy(f + 1, 1 - par).wait()
                gat_copy(1 - par).start()

            gat_copy(par).wait()

            @pl.when(g + 2 < G)
            def _():
                idx_copy(f + 2, par).start()

            @pl.when(g >= 2)
            def _():
                out_copy(f - 2, par).wait()

            s, _ = chunk_sb(f)
            gb = gbuf.at[par]
            ob = obuf.at[par]
            ssplat = jnp.full((16,), 0, jnp.int32) + s

            @plsc.parallel_loop(0, N_FEATURES, unroll=2)
            def _(i):
                d = lax.rem(i, 16)
                f0 = i - d
                col = ((iota16 + d) & 15) + f0
                pvec = plsc.load_gather(pos_v, [ssplat, col])
                for b0 in range(0, C, 16):
                    row = iota16 + b0
                    vec = plsc.load_gather(gb, [row, col])
                    plsc.store_scatter(ob, [col, row], vec + pvec)

            out_copy(f, par).start()

    out_copy(f0 + G - 2, 0).wait()
    out_copy(f0 + G - 1, 1).wait()


@jax.jit
def _embed(xt, tblt, post, tail):
    run = pl.kernel(
        _body,
        out_type=(
            jax.ShapeDtypeStruct((SEQ_LEN, N_FEATURES, BATCH), jnp.float32),
            jax.ShapeDtypeStruct((NC, VPAD, ROW_W), jnp.float32),
        ),
        mesh=plsc.VectorSubcoreMesh(core_axis_name="c", subcore_axis_name="s"),
        scratch_types=[
            pltpu.VMEM((N_FEATURES, ROW_W), jnp.float32),
            pltpu.VMEM((ROW_W, ROW_W), jnp.float32),
            pltpu.VMEM((SEQ_LEN, N_FEATURES), jnp.float32),
            pltpu.VMEM((2, C), jnp.int32),
            pltpu.VMEM((2, C, ROW_W), jnp.float32),
            pltpu.VMEM((2, N_FEATURES, C), jnp.float32),
            pltpu.SemaphoreType.DMA,
            pltpu.SemaphoreType.DMA,
            pltpu.SemaphoreType.DMA,
            pltpu.SemaphoreType.DMA,
            pltpu.SemaphoreType.DMA,
        ],
        compiler_params=pltpu.CompilerParams(
            use_tc_tiling_on_sc=True, needs_layout_passes=False),
    )
    out_t, _ = run(xt, tblt, post, tail)
    return out_t


def kernel(x, text_embedding_weight, pos_embedding):
    bs, seq_len = x.shape
    xt = x.T.astype(jnp.int32)
    tblt = text_embedding_weight.T
    post = pos_embedding.reshape(-1, pos_embedding.shape[-1])[:seq_len]
    tail = jnp.pad(text_embedding_weight[VBLK_FULL * ROW_W:],
                   ((0, 0), (0, ROW_W - N_FEATURES)))
    out_t = _embed(xt, tblt, post, tail)
    return jnp.transpose(out_t, (2, 0, 1))


# speedup vs baseline: 5.2791x; 1.1599x over previous
import jax
import jax.numpy as jnp
from jax import lax
from jax.experimental import pallas as pl
from jax.experimental.pallas import tpu as pltpu
from jax.experimental.pallas import tpu_sc as plsc

N_FEATURES = 64
BATCH = 4096
SEQ_LEN = 200
VOCAB_ROWS = 100000
VPAD = 100096
ROW_W = 128

NC = 2
NS = 16
NW = NC * NS

C = 128
NTB = BATCH // C
G = SEQ_LEN * NTB // NW

VBLKS = VPAD // ROW_W
VBLK_FULL = VOCAB_ROWS // ROW_W
VTAIL = VOCAB_ROWS - VBLK_FULL * ROW_W
BLK_PER_W = (VBLKS + NW - 1) // NW


def _transpose_body(tblt_ref, tail_ref, scr_ref, t_in, t_out):
    cid = lax.axis_index("c")
    sid = lax.axis_index("s")
    iota16 = lax.iota(jnp.int32, 16)
    wid = sid * NC + cid

    def transpose_block(v0):
        @plsc.parallel_loop(0, N_FEATURES, unroll=2)
        def _(i):
            d = lax.rem(i, 16)
            f0 = i - d
            frow = ((iota16 + d) & 15) + f0
            for w0 in range(0, ROW_W, 16):
                wcol = iota16 + w0
                vec = plsc.load_gather(t_in, [frow, wcol])
                plsc.store_scatter(t_out, [wcol, frow], vec)
        pltpu.sync_copy(t_out, scr_ref.at[pl.ds(v0, ROW_W)])

    @pl.loop(0, BLK_PER_W)
    def _(g):
        blk = wid * BLK_PER_W + g

        @pl.when(blk < VBLK_FULL)
        def _():
            v0 = blk * ROW_W
            pltpu.sync_copy(tblt_ref.at[:, pl.ds(v0, ROW_W)], t_in)
            transpose_block(v0)

        @pl.when(blk == VBLK_FULL)
        def _():
            v0 = VBLK_FULL * ROW_W
            pltpu.sync_copy(tail_ref, t_out.at[pl.ds(0, VTAIL)])
            pltpu.sync_copy(t_out.at[pl.ds(0, VTAIL)],
                            scr_ref.at[pl.ds(v0, VTAIL)])


def _gather_body(xt_ref, my_rm, post_ref, out_ref, pos_v,
                 idxb, gbuf, obuf, sem_g1, sem_idx, sem_gat, sem_o0, sem_o1):
    cid = lax.axis_index("c")
    sid = lax.axis_index("s")
    iota16 = lax.iota(jnp.int32, 16)

    pltpu.sync_copy(post_ref, pos_v)

    wid = sid * NC + cid
    f0 = wid * G

    def chunk_sb(f):
        return f // NTB, lax.rem(f, NTB)

    def idx_copy(f, b):
        s, tb = chunk_sb(f)
        return pltpu.make_async_copy(xt_ref.at[s, pl.ds(tb * C, C)],
                                     idxb.at[b], sem_idx)

    def gat_copy(b):
        return pltpu.make_async_copy(my_rm.at[idxb.at[b]], gbuf.at[b],
                                     sem_g1 if b else sem_gat)

    def out_copy(f, b):
        s, tb = chunk_sb(f)
        return pltpu.make_async_copy(
            obuf.at[b], out_ref.at[s, :, pl.ds(tb * C, C)],
            sem_o1 if b else sem_o0)

    idx_copy(f0, 0).start()
    idx_copy(f0, 0).wait()
    gat_copy(0).start()
    idx_copy(f0 + 1, 1).start()

    @pl.loop(0, G // 2)
    def _(go):
        for par in range(2):
            g = go * 2 + par
            f = f0 + g

            @pl.when(g + 1 < G)
            def _():
                idx_copy(f + 1, 1 - par).wait()
                gat_copy(1 - par).start()

            gat_copy(par).wait()

            @pl.when(g + 2 < G)
            def _():
                idx_copy(f + 2, par).start()

            @pl.when(g >= 2)
            def _():
                out_copy(f - 2, par).wait()

            s, _ = chunk_sb(f)
            gb = gbuf.at[par]
            ob = obuf.at[par]
            ssplat = jnp.full((16,), 0, jnp.int32) + s

            @plsc.parallel_loop(0, N_FEATURES, unroll=2)
            def _(i):
                d = lax.rem(i, 16)
                f0 = i - d
                col = ((iota16 + d) & 15) + f0
                pvec = plsc.load_gather(pos_v, [ssplat, col])
                for b0 in range(0, C, 16):
                    row = iota16 + b0
                    vec = plsc.load_gather(gb, [row, col])
                    plsc.store_scatter(ob, [col, row], vec + pvec)

            out_copy(f, par).start()

    out_copy(f0 + G - 2, 0).wait()
    out_copy(f0 + G - 1, 1).wait()


_PARAMS = pltpu.CompilerParams(
    use_tc_tiling_on_sc=True, needs_layout_passes=False)
_MESH = dict(core_axis_name="c", subcore_axis_name="s")


@jax.jit
def _embed(xt, tblt, post, tail):
    run_t = pl.kernel(
        _transpose_body,
        out_type=jax.ShapeDtypeStruct((VPAD, ROW_W), jnp.float32),
        mesh=plsc.VectorSubcoreMesh(**_MESH),
        scratch_types=[
            pltpu.VMEM((N_FEATURES, ROW_W), jnp.float32),
            pltpu.VMEM((ROW_W, ROW_W), jnp.float32),
        ],
        compiler_params=_PARAMS,
    )
    scr = run_t(tblt, tail)
    run_g = pl.kernel(
        _gather_body,
        out_type=jax.ShapeDtypeStruct((SEQ_LEN, N_FEATURES, BATCH),
                                      jnp.float32),
        mesh=plsc.VectorSubcoreMesh(**_MESH),
        scratch_types=[
            pltpu.VMEM((SEQ_LEN, N_FEATURES), jnp.float32),
            pltpu.VMEM((2, C), jnp.int32),
            pltpu.VMEM((2, C, ROW_W), jnp.float32),
            pltpu.VMEM((2, N_FEATURES, C), jnp.float32),
            pltpu.SemaphoreType.DMA,
            pltpu.SemaphoreType.DMA,
            pltpu.SemaphoreType.DMA,
            pltpu.SemaphoreType.DMA,
            pltpu.SemaphoreType.DMA,
        ],
        compiler_params=_PARAMS,
    )
    return run_g(xt, scr, post)


def kernel(x, text_embedding_weight, pos_embedding):
    bs, seq_len = x.shape
    xt = x.T.astype(jnp.int32)
    tblt = text_embedding_weight.T
    post = pos_embedding.reshape(-1, pos_embedding.shape[-1])[:seq_len]
    tail = jnp.pad(text_embedding_weight[VBLK_FULL * ROW_W:],
                   ((0, 0), (0, ROW_W - N_FEATURES)))
    out_t = _embed(xt, tblt, post, tail)
    return jnp.transpose(out_t, (2, 0, 1))


# speedup vs baseline: 5.7258x; 1.0846x over previous
import jax
import jax.numpy as jnp
from jax import lax
from jax.experimental import pallas as pl
from jax.experimental.pallas import tpu as pltpu
from jax.experimental.pallas import tpu_sc as plsc

N_FEATURES = 64
BATCH = 4096
SEQ_LEN = 200
VOCAB_ROWS = 100000
VPAD = 100096
ROW_W = 128

NC = 2
NS = 16
NW = NC * NS

C = 128
CW = 256
NTBW = BATCH // CW
GW = SEQ_LEN * NTBW // NW

VBLKS = VPAD // ROW_W
VBLK_FULL = VOCAB_ROWS // ROW_W
VTAIL = VOCAB_ROWS - VBLK_FULL * ROW_W
BLK_PER_W = (VBLKS + NW - 1) // NW


def _transpose_body(tblt_ref, tail_ref, scr_ref, t_in, t_out):
    cid = lax.axis_index("c")
    sid = lax.axis_index("s")
    iota16 = lax.iota(jnp.int32, 16)
    wid = sid * NC + cid

    def transpose_block(v0):
        @plsc.parallel_loop(0, N_FEATURES, unroll=2)
        def _(i):
            d = lax.rem(i, 16)
            f0 = i - d
            frow = ((iota16 + d) & 15) + f0
            for w0 in range(0, ROW_W, 16):
                wcol = iota16 + w0
                vec = plsc.load_gather(t_in, [frow, wcol])
                plsc.store_scatter(t_out, [wcol, frow], vec)
        pltpu.sync_copy(t_out, scr_ref.at[pl.ds(v0, ROW_W)])

    @pl.loop(0, BLK_PER_W)
    def _(g):
        blk = wid * BLK_PER_W + g

        @pl.when(blk < VBLK_FULL)
        def _():
            v0 = blk * ROW_W
            pltpu.sync_copy(tblt_ref.at[:, pl.ds(v0, ROW_W)], t_in)
            transpose_block(v0)

        @pl.when(blk == VBLK_FULL)
        def _():
            v0 = VBLK_FULL * ROW_W
            pltpu.sync_copy(tail_ref, t_out.at[pl.ds(0, VTAIL)])
            pltpu.sync_copy(t_out.at[pl.ds(0, VTAIL)],
                            scr_ref.at[pl.ds(v0, VTAIL)])


def _gather_body(xt_ref, my_rm, post_ref, out_ref, pos_v,
                 idxb, gbuf, obuf, sem_g1, sem_idx, sem_gat, sem_o0, sem_o1):
    cid = lax.axis_index("c")
    sid = lax.axis_index("s")
    iota16 = lax.iota(jnp.int32, 16)

    pltpu.sync_copy(post_ref, pos_v)

    wid = sid * NC + cid
    f0 = wid * GW

    def chunk_sb(f):
        return f // NTBW, lax.rem(f, NTBW)

    def idx_copy(f, b):
        s, tb = chunk_sb(f)
        return pltpu.make_async_copy(xt_ref.at[s, pl.ds(tb * CW, CW)],
                                     idxb.at[b], sem_idx)

    def gat_copies(b):
        sem = sem_g1 if b else sem_gat
        return [
            pltpu.make_async_copy(my_rm.at[idxb.at[b, pl.ds(k * C, C)]],
                                  gbuf.at[b, pl.ds(k * C, C)], sem)
            for k in range(CW // C)
        ]

    def out_copy(f, b):
        s, tb = chunk_sb(f)
        return pltpu.make_async_copy(
            obuf.at[b], out_ref.at[s, :, pl.ds(tb * CW, CW)],
            sem_o1 if b else sem_o0)

    idx_copy(f0, 0).start()
    idx_copy(f0, 0).wait()
    for cp in gat_copies(0):
        cp.start()
    idx_copy(f0 + 1, 1).start()

    @pl.loop(0, GW // 2)
    def _(go):
        for par in range(2):
            g = go * 2 + par
            f = f0 + g

            @pl.when(g + 1 < GW)
            def _():
                idx_copy(f + 1, 1 - par).wait()
                for cp in gat_copies(1 - par):
                    cp.start()

            for cp in gat_copies(par):
                cp.wait()

            @pl.when(g + 2 < GW)
            def _():
                idx_copy(f + 2, par).start()

            @pl.when(g >= 2)
            def _():
                out_copy(f - 2, par).wait()

            s, _ = chunk_sb(f)
            gb = gbuf.at[par]
            ob = obuf.at[par]
            ssplat = jnp.full((16,), 0, jnp.int32) + s

            @plsc.parallel_loop(0, N_FEATURES, unroll=2)
            def _(i):
                d = lax.rem(i, 16)
                f0 = i - d
                col = ((iota16 + d) & 15) + f0
                pvec = plsc.load_gather(pos_v, [ssplat, col])
                for b0 in range(0, CW, 16):
                    row = iota16 + b0
                    vec = plsc.load_gather(gb, [row, col])
                    plsc.store_scatter(ob, [col, row], vec + pvec)

            out_copy(f, par).start()

    out_copy(f0 + GW - 2, 0).wait()
    out_copy(f0 + GW - 1, 1).wait()


_PARAMS = pltpu.CompilerParams(
    use_tc_tiling_on_sc=True, needs_layout_passes=False)
_MESH = dict(core_axis_name="c", subcore_axis_name="s")


@jax.jit
def _embed(xt, tblt, post, tail):
    run_t = pl.kernel(
        _transpose_body,
        out_type=jax.ShapeDtypeStruct((VPAD, ROW_W), jnp.float32),
        mesh=plsc.VectorSubcoreMesh(**_MESH),
        scratch_types=[
            pltpu.VMEM((N_FEATURES, ROW_W), jnp.float32),
            pltpu.VMEM((ROW_W, ROW_W), jnp.float32),
        ],
        compiler_params=_PARAMS,
    )
    scr = run_t(tblt, tail)
    run_g = pl.kernel(
        _gather_body,
        out_type=jax.ShapeDtypeStruct((SEQ_LEN, N_FEATURES, BATCH),
                                      jnp.float32),
        mesh=plsc.VectorSubcoreMesh(**_MESH),
        scratch_types=[
            pltpu.VMEM((SEQ_LEN, N_FEATURES), jnp.float32),
            pltpu.VMEM((2, CW), jnp.int32),
            pltpu.VMEM((2, CW, ROW_W), jnp.float32),
            pltpu.VMEM((2, N_FEATURES, CW), jnp.float32),
            pltpu.SemaphoreType.DMA,
            pltpu.SemaphoreType.DMA,
            pltpu.SemaphoreType.DMA,
            pltpu.SemaphoreType.DMA,
            pltpu.SemaphoreType.DMA,
        ],
        compiler_params=_PARAMS,
    )
    return run_g(xt, scr, post)


def kernel(x, text_embedding_weight, pos_embedding):
    bs, seq_len = x.shape
    xt = x.T.astype(jnp.int32)
    tblt = text_embedding_weight.T
    post = pos_embedding.reshape(-1, pos_embedding.shape[-1])[:seq_len]
    tail = jnp.pad(text_embedding_weight[VBLK_FULL * ROW_W:],
                   ((0, 0), (0, ROW_W - N_FEATURES)))
    out_t = _embed(xt, tblt, post, tail)
    return jnp.transpose(out_t, (2, 0, 1))
